# baseline (device time: 12553 ns/iter reference)
import jax
import jax.numpy as jnp
from jax import lax
from jax.experimental import pallas as pl
from jax.experimental.pallas import tpu as pltpu

N_DEV = 8


def kernel(x, w_mat):
    m_per, k = x.shape
    n = w_mat.shape[1]
    n_per = n // N_DEV
    m_total = N_DEV * m_per

    def gelu(y):
        c = 0.7978845608028654
        return 0.5 * y * (1.0 + jnp.tanh(c * (y + 0.044715 * y * y * y)))

    def body(x_ref, w_ref, out_ref, send_buf, recv_buf, send_sems, recv_sems):
        my = lax.axis_index("i")

        barrier = pltpu.get_barrier_semaphore()
        for j in range(N_DEV):
            @pl.when(my != j)
            def _():
                pl.semaphore_signal(
                    barrier, inc=1, device_id=(j,),
                    device_id_type=pl.DeviceIdType.MESH)

        x_val = x_ref[:, :]
        for d in range(1, N_DEV):
            t = (my + d) % N_DEV
            w_blk = w_ref[:, pl.ds(t * n_per, n_per)]
            yb = gelu(jnp.dot(x_val, w_blk,
                              preferred_element_type=jnp.float32))
            send_buf[t, :, :] = yb.astype(jnp.bfloat16)
            if d == 1:
                pl.semaphore_wait(barrier, N_DEV - 1)
            pltpu.make_async_remote_copy(
                src_ref=send_buf.at[t],
                dst_ref=recv_buf.at[my],
                send_sem=send_sems.at[t],
                recv_sem=recv_sems.at[my],
                device_id=(t,),
                device_id_type=pl.DeviceIdType.MESH,
            ).start()

        w_own = w_ref[:, pl.ds(my * n_per, n_per)]
        y_own = gelu(jnp.dot(x_val, w_own, preferred_element_type=jnp.float32))
        out_ref[pl.ds(my * m_per, m_per), :] = (
            y_own.astype(jnp.bfloat16).astype(jnp.float32))

        for d in range(1, N_DEV):
            s = (my - d) % N_DEV
            pltpu.make_async_remote_copy(
                src_ref=send_buf.at[s],
                dst_ref=recv_buf.at[s],
                send_sem=send_sems.at[s],
                recv_sem=recv_sems.at[s],
                device_id=(s,),
                device_id_type=pl.DeviceIdType.MESH,
            ).wait_recv()
            out_ref[pl.ds(s * m_per, m_per), :] = (
                recv_buf[s, :, :].astype(jnp.float32))

        for d in range(1, N_DEV):
            t = (my + d) % N_DEV
            pltpu.make_async_remote_copy(
                src_ref=send_buf.at[t],
                dst_ref=recv_buf.at[my],
                send_sem=send_sems.at[t],
                recv_sem=recv_sems.at[my],
                device_id=(t,),
                device_id_type=pl.DeviceIdType.MESH,
            ).wait_send()

    return pl.pallas_call(
        body,
        out_shape=jax.ShapeDtypeStruct((m_total, n_per), jnp.float32),
        in_specs=[pl.BlockSpec(memory_space=pltpu.VMEM),
                  pl.BlockSpec(memory_space=pltpu.VMEM)],
        out_specs=pl.BlockSpec(memory_space=pltpu.VMEM),
        scratch_shapes=[
            pltpu.VMEM((N_DEV, m_per, n_per), jnp.bfloat16),
            pltpu.VMEM((N_DEV, m_per, n_per), jnp.bfloat16),
            pltpu.SemaphoreType.DMA((N_DEV,)),
            pltpu.SemaphoreType.DMA((N_DEV,)),
        ],
        compiler_params=pltpu.CompilerParams(collective_id=0),
    )(x, w_mat)
